# baseline (speedup 1.0000x reference)
import functools

import jax
import jax.numpy as jnp
import numpy as np
from jax.experimental import pallas as pl

NUM_BINS = 15
_EDGES = tuple(float(x) for x in np.asarray(jnp.linspace(0.0, 1.0, NUM_BINS + 1)))


def _calib_kernel(pro_ref, lab_ref, cnt_ref, csum_ref, asum_ref):
    step = pl.program_id(0)

    @pl.when(step == 0)
    def _init():
        cnt_ref[...] = jnp.zeros_like(cnt_ref)
        csum_ref[...] = jnp.zeros_like(csum_ref)
        asum_ref[...] = jnp.zeros_like(asum_ref)

    p = pro_ref[...]
    bn, c = p.shape
    conf = jnp.max(p, axis=1)
    col = jax.lax.broadcasted_iota(jnp.int32, (bn, c), 1)
    pred = jnp.min(jnp.where(p == conf[:, None], col, c), axis=1)
    lab = lab_ref[0, 0, :]
    corr = (pred == lab).astype(jnp.float32)

    cnt_lt = jnp.zeros((bn,), jnp.int32)
    on_edge = jnp.zeros((bn,), jnp.bool_)
    for e in _EDGES:
        cnt_lt += (conf > e).astype(jnp.int32)
        on_edge |= conf == e
    binid = cnt_lt - 1
    invalid = on_edge | (binid < 0) | (binid >= NUM_BINS)
    binid = jnp.where(invalid, NUM_BINS, binid)

    bidx = jax.lax.broadcasted_iota(jnp.int32, (bn, NUM_BINS + 1), 1)
    m = (binid[:, None] == bidx).astype(jnp.float32)
    cnt_ref[0, :] += jnp.sum(m, axis=0)
    csum_ref[0, :] += jnp.sum(conf[:, None] * m, axis=0)
    asum_ref[0, :] += jnp.sum(corr[:, None] * m, axis=0)


@jax.jit
def kernel(probas, labels):
    n, c = probas.shape
    bn = 8000 if n % 8000 == 0 else n
    nb = n // bn
    labels3 = labels.reshape(nb, 1, bn)

    out_shape = jax.ShapeDtypeStruct((1, NUM_BINS + 1), jnp.float32)
    out_spec = pl.BlockSpec((1, NUM_BINS + 1), lambda i: (0, 0))
    cnt, csum, asum = pl.pallas_call(
        _calib_kernel,
        grid=(nb,),
        in_specs=[
            pl.BlockSpec((bn, c), lambda i: (i, 0)),
            pl.BlockSpec((1, 1, bn), lambda i: (i, 0, 0)),
        ],
        out_specs=[out_spec, out_spec, out_spec],
        out_shape=[out_shape, out_shape, out_shape],
    )(probas, labels3)

    counts = cnt[0, :NUM_BINS]
    denom = jnp.maximum(counts, 1.0)
    empty = counts == 0.0
    confs = jnp.where(empty, jnp.nan, csum[0, :NUM_BINS] / denom)
    accs = jnp.where(empty, jnp.nan, asum[0, :NUM_BINS] / denom)
    return confs, accs, counts
